# baseline (speedup 1.0000x reference)
import jax
import jax.numpy as jnp
from jax.experimental import pallas as pl
from jax.experimental.pallas import tpu as pltpu

_N = 4096
_D = 512
_BM = 512
_F8 = jnp.float8_e4m3fn
_SCALE = 64.0
_INV_SCALE = 1.0 / _SCALE


def _split_f8(x):
    hi = x.astype(_F8)
    lo = ((x - hi.astype(jnp.float32)) * _SCALE).astype(_F8)
    return hi, lo


def _gcn_body(h_ref, w_ref, adj_ref, b_ref, out_ref, s1_ref, s2_ref):
    i = pl.program_id(0)

    @pl.when(i == 0)
    def _support():
        hb = h_ref[...].astype(jnp.bfloat16)
        wb = w_ref[...].astype(jnp.bfloat16)
        sup = jnp.dot(hb, wb, preferred_element_type=jnp.float32)
        s1, s2 = _split_f8(sup)
        s1_ref[...] = s1
        s2_ref[...] = s2

    @pl.when(i > 0)
    def _rows():
        a1, a2 = _split_f8(adj_ref[...])
        p0 = jnp.dot(a1, s1_ref[...], preferred_element_type=jnp.float32)
        p1 = jnp.dot(a1, s2_ref[...], preferred_element_type=jnp.float32)
        p2 = jnp.dot(a2, s1_ref[...], preferred_element_type=jnp.float32)
        acc = p0 + (p1 + p2) * _INV_SCALE
        out_ref[...] = jnp.maximum(acc + b_ref[...], 0.0)


def kernel(h, adj, W, b):
    b2 = b.reshape(1, _D)
    row = lambda i: (jnp.maximum(i - 1, 0), 0)
    return pl.pallas_call(
        _gcn_body,
        grid=(_N // _BM + 1,),
        in_specs=[
            pl.BlockSpec((_N, _D), lambda i: (0, 0)),
            pl.BlockSpec((_D, _D), lambda i: (0, 0)),
            pl.BlockSpec((_BM, _N), row),
            pl.BlockSpec((1, _D), lambda i: (0, 0)),
        ],
        out_specs=pl.BlockSpec((_BM, _D), row),
        out_shape=jax.ShapeDtypeStruct((_N, _D), jnp.float32),
        scratch_shapes=[
            pltpu.VMEM((_N, _D), _F8),
            pltpu.VMEM((_N, _D), _F8),
        ],
        compiler_params=pltpu.CompilerParams(
            dimension_semantics=("arbitrary",),
        ),
    )(h, W, adj, b2)


# speedup vs baseline: 1.0389x; 1.0389x over previous
import jax
import jax.numpy as jnp
from jax.experimental import pallas as pl
from jax.experimental.pallas import tpu as pltpu

_N = 4096
_D = 512
_BM = 512
_F8 = jnp.float8_e4m3fn
_SCALE = 64.0
_INV_SCALE = 1.0 / _SCALE


def _split_f8(x):
    xb = x.astype(jnp.bfloat16)
    hi = xb.astype(_F8)
    lo = ((xb - hi.astype(jnp.bfloat16)) * jnp.bfloat16(_SCALE)).astype(_F8)
    return hi, lo


def _gcn_body(h_ref, w_ref, adj_ref, b_ref, out_ref, s1_ref, s2_ref):
    i = pl.program_id(0)

    @pl.when(i == 0)
    def _support():
        hb = h_ref[...].astype(jnp.bfloat16)
        wb = w_ref[...].astype(jnp.bfloat16)
        sup = jnp.dot(hb, wb, preferred_element_type=jnp.float32)
        s1, s2 = _split_f8(sup)
        s1_ref[...] = s1
        s2_ref[...] = s2

    @pl.when(i > 0)
    def _rows():
        a1, a2 = _split_f8(adj_ref[...])
        p0 = jnp.dot(a1, s1_ref[...], preferred_element_type=jnp.float32)
        p1 = jnp.dot(a1, s2_ref[...], preferred_element_type=jnp.float32)
        p2 = jnp.dot(a2, s1_ref[...], preferred_element_type=jnp.float32)
        acc = p0 + (p1 + p2) * _INV_SCALE
        out_ref[...] = jnp.maximum(acc + b_ref[...], 0.0)


def kernel(h, adj, W, b):
    b2 = b.reshape(1, _D)
    row = lambda i: (jnp.maximum(i - 1, 0), 0)
    return pl.pallas_call(
        _gcn_body,
        grid=(_N // _BM + 1,),
        in_specs=[
            pl.BlockSpec((_N, _D), lambda i: (0, 0)),
            pl.BlockSpec((_D, _D), lambda i: (0, 0)),
            pl.BlockSpec((_BM, _N), row),
            pl.BlockSpec((1, _D), lambda i: (0, 0)),
        ],
        out_specs=pl.BlockSpec((_BM, _D), row),
        out_shape=jax.ShapeDtypeStruct((_N, _D), jnp.float32),
        scratch_shapes=[
            pltpu.VMEM((_N, _D), _F8),
            pltpu.VMEM((_N, _D), _F8),
        ],
        compiler_params=pltpu.CompilerParams(
            dimension_semantics=("arbitrary",),
        ),
    )(h, W, adj, b2)
